# baseline (speedup 1.0000x reference)
import functools

import jax
import jax.numpy as jnp
from jax.experimental import pallas as pl
from jax.experimental.pallas import tpu as pltpu

_EPS = 1e-5
_VMEM_LIMIT = 56 * 1024 * 1024
_BLK = 1


def _row_shift(src, dy, W):
    if dy == 0:
        return src
    c, HW = src.shape
    z = jnp.zeros((c, W), src.dtype)
    if dy > 0:
        return jnp.concatenate([src[:, W:], z], axis=1)
    return jnp.concatenate([z, src[:, :HW - W]], axis=1)


def _build_taps(src, cm, taps_ref, W):
    c, HW = src.shape
    for r, dy in enumerate((-1, 0, 1)):
        base = _row_shift(src, dy, W)
        tm = jnp.concatenate([base[:, HW - 1:], base[:, :HW - 1]], axis=1) * cm[0:1, :]
        tp = jnp.concatenate([base[:, 1:], base[:, :1]], axis=1) * cm[1:2, :]
        taps_ref[(3 * r) * c:(3 * r + 1) * c, :] = tm
        taps_ref[(3 * r + 1) * c:(3 * r + 2) * c, :] = base
        taps_ref[(3 * r + 2) * c:(3 * r + 3) * c, :] = tp


def _moments(h):
    return (jnp.sum(h, axis=1, keepdims=True),
            jnp.sum(h * h, axis=1, keepdims=True))


def _conv1_body(W, x_ref, w_ref, b_ref, cm_ref, h1_ref, s_ref, q_ref, taps):
    cin = x_ref.shape[1]
    for i in range(_BLK):
        xb = x_ref[i].astype(jnp.bfloat16).reshape(cin, -1)
        _build_taps(xb, cm_ref[...], taps.at[i], W)
        h = jnp.dot(w_ref[...], taps.at[i][...],
                    preferred_element_type=jnp.float32) + b_ref[...]
        h1_ref[i] = h.astype(jnp.bfloat16)
        s, q = _moments(h)
        s_ref[i] = s
        q_ref[i] = q


def _conv2_body(W, h1_ref, sc_ref, sh_ref, w_ref, b_ref, cm_ref,
                h2_ref, s_ref, q_ref, taps):
    sc = sc_ref[...]
    sh = sh_ref[...]
    for i in range(_BLK):
        a32 = jnp.maximum(h1_ref[i].astype(jnp.float32) * sc + sh, 0.0)
        _build_taps(a32.astype(jnp.bfloat16), cm_ref[...], taps.at[i], W)
        h = jnp.dot(w_ref[...], taps.at[i][...],
                    preferred_element_type=jnp.float32) + b_ref[...]
        h2_ref[i] = h.astype(jnp.bfloat16)
        s, q = _moments(h)
        s_ref[i] = s
        q_ref[i] = q


def _fold(sum_p, sq_p, gamma, beta, count):
    n = sum_p.shape[0]
    denom = jnp.float32(n * count)
    mu = jnp.sum(sum_p[:, :, 0], axis=0) / denom
    var = jnp.sum(sq_p[:, :, 0], axis=0) / denom - mu * mu
    scale = gamma.astype(jnp.float32) * jax.lax.rsqrt(var + _EPS)
    shift = beta.astype(jnp.float32) - mu * scale
    return scale[:, None], shift[:, None]


def _flatten_w(w):
    cout, cin = w.shape[0], w.shape[1]
    w2d = jnp.transpose(w.astype(jnp.float32).reshape(cout, cin, 9), (0, 2, 1))
    return w2d.reshape(cout, 9 * cin).astype(jnp.bfloat16)


@jax.jit
def kernel(x, w1, b1, gamma1, beta1, w2, b2, gamma2, beta2):
    N, Cin, H, W = x.shape
    HW = H * W
    Cmid, Cout = w1.shape[0], w2.shape[0]
    bf16 = jnp.bfloat16

    w1_2d = _flatten_w(w1)
    w2_2d = _flatten_w(w2)
    b1c = b1.astype(jnp.float32).reshape(Cmid, 1)
    b2c = b2.astype(jnp.float32).reshape(Cout, 1)

    col = jnp.arange(HW, dtype=jnp.int32) % W
    cm = jnp.stack([col >= 1, col < W - 1], axis=0).astype(bf16)

    cparams = pltpu.CompilerParams(
        dimension_semantics=("arbitrary",),
        vmem_limit_bytes=_VMEM_LIMIT,
    )

    def img_spec(C):
        return pl.BlockSpec((_BLK, C, HW), lambda n: (n, 0, 0))

    def img4_spec(C):
        return pl.BlockSpec((_BLK, C, H, W), lambda n: (n, 0, 0, 0))

    def full_spec(shape):
        return pl.BlockSpec(shape, lambda n: (0,) * len(shape))

    def stat_spec(C):
        return pl.BlockSpec((_BLK, C, 1), lambda n: (n, 0, 0))

    h1, sum1, sq1 = pl.pallas_call(
        functools.partial(_conv1_body, W),
        grid=(N // _BLK,),
        in_specs=[img4_spec(Cin), full_spec((Cmid, 9 * Cin)),
                  full_spec((Cmid, 1)), full_spec((2, HW))],
        out_specs=[img_spec(Cmid), stat_spec(Cmid), stat_spec(Cmid)],
        out_shape=[jax.ShapeDtypeStruct((N, Cmid, HW), bf16),
                   jax.ShapeDtypeStruct((N, Cmid, 1), jnp.float32),
                   jax.ShapeDtypeStruct((N, Cmid, 1), jnp.float32)],
        scratch_shapes=[pltpu.VMEM((_BLK, 9 * Cin, HW), bf16)],
        compiler_params=cparams,
    )(x, w1_2d, b1c, cm)

    sc1, sh1 = _fold(sum1, sq1, gamma1, beta1, HW)

    h2, sum2, sq2 = pl.pallas_call(
        functools.partial(_conv2_body, W),
        grid=(N // _BLK,),
        in_specs=[img_spec(Cmid), full_spec((Cmid, 1)), full_spec((Cmid, 1)),
                  full_spec((Cout, 9 * Cmid)), full_spec((Cout, 1)),
                  full_spec((2, HW))],
        out_specs=[img_spec(Cout), stat_spec(Cout), stat_spec(Cout)],
        out_shape=[jax.ShapeDtypeStruct((N, Cout, HW), bf16),
                   jax.ShapeDtypeStruct((N, Cout, 1), jnp.float32),
                   jax.ShapeDtypeStruct((N, Cout, 1), jnp.float32)],
        scratch_shapes=[pltpu.VMEM((_BLK, 9 * Cmid, HW), bf16)],
        compiler_params=cparams,
    )(h1, sc1, sh1, w2_2d, b2c, cm)

    sc2, sh2 = _fold(sum2, sq2, gamma2, beta2, HW)
    y = jnp.maximum(h2.astype(jnp.float32) * sc2[None, :, :] + sh2[None, :, :], 0.0)
    return y.reshape(N, Cout, H, W)


# speedup vs baseline: 1.0373x; 1.0373x over previous
import functools

import jax
import jax.numpy as jnp
from jax.experimental import pallas as pl
from jax.experimental.pallas import tpu as pltpu

_EPS = 1e-5
_VMEM_LIMIT = 56 * 1024 * 1024
_BLK = 4


def _row_shift(src, dy, W):
    if dy == 0:
        return src
    c, HW = src.shape
    z = jnp.zeros((c, W), src.dtype)
    if dy > 0:
        return jnp.concatenate([src[:, W:], z], axis=1)
    return jnp.concatenate([z, src[:, :HW - W]], axis=1)


def _build_taps(src, cm, taps_ref, W):
    c, HW = src.shape
    for r, dy in enumerate((-1, 0, 1)):
        base = _row_shift(src, dy, W)
        tm = jnp.concatenate([base[:, HW - 1:], base[:, :HW - 1]], axis=1) * cm[0:1, :]
        tp = jnp.concatenate([base[:, 1:], base[:, :1]], axis=1) * cm[1:2, :]
        taps_ref[(3 * r) * c:(3 * r + 1) * c, :] = tm
        taps_ref[(3 * r + 1) * c:(3 * r + 2) * c, :] = base
        taps_ref[(3 * r + 2) * c:(3 * r + 3) * c, :] = tp


def _moments(h):
    return (jnp.sum(h, axis=1, keepdims=True),
            jnp.sum(h * h, axis=1, keepdims=True))


def _conv1_body(W, x_ref, w_ref, b_ref, cm_ref, h1_ref, s_ref, q_ref, taps):
    cin = x_ref.shape[1]
    for i in range(_BLK):
        xb = x_ref[i].astype(jnp.bfloat16).reshape(cin, -1)
        _build_taps(xb, cm_ref[...], taps.at[i], W)
        h = jnp.dot(w_ref[...], taps.at[i][...],
                    preferred_element_type=jnp.float32) + b_ref[...]
        h1_ref[i] = h.astype(jnp.bfloat16)
        s, q = _moments(h)
        s_ref[i] = s
        q_ref[i] = q


def _conv2_body(W, h1_ref, sc_ref, sh_ref, w_ref, b_ref, cm_ref,
                h2_ref, s_ref, q_ref, taps):
    sc = sc_ref[...]
    sh = sh_ref[...]
    for i in range(_BLK):
        a32 = jnp.maximum(h1_ref[i].astype(jnp.float32) * sc + sh, 0.0)
        _build_taps(a32.astype(jnp.bfloat16), cm_ref[...], taps.at[i], W)
        h = jnp.dot(w_ref[...], taps.at[i][...],
                    preferred_element_type=jnp.float32) + b_ref[...]
        h2_ref[i] = h.astype(jnp.bfloat16)
        s, q = _moments(h)
        s_ref[i] = s
        q_ref[i] = q


def _fold(sum_p, sq_p, gamma, beta, count):
    n = sum_p.shape[0]
    denom = jnp.float32(n * count)
    mu = jnp.sum(sum_p[:, :, 0], axis=0) / denom
    var = jnp.sum(sq_p[:, :, 0], axis=0) / denom - mu * mu
    scale = gamma.astype(jnp.float32) * jax.lax.rsqrt(var + _EPS)
    shift = beta.astype(jnp.float32) - mu * scale
    return scale[:, None], shift[:, None]


def _flatten_w(w):
    cout, cin = w.shape[0], w.shape[1]
    w2d = jnp.transpose(w.astype(jnp.float32).reshape(cout, cin, 9), (0, 2, 1))
    return w2d.reshape(cout, 9 * cin).astype(jnp.bfloat16)


@jax.jit
def kernel(x, w1, b1, gamma1, beta1, w2, b2, gamma2, beta2):
    N, Cin, H, W = x.shape
    HW = H * W
    Cmid, Cout = w1.shape[0], w2.shape[0]
    bf16 = jnp.bfloat16

    w1_2d = _flatten_w(w1)
    w2_2d = _flatten_w(w2)
    b1c = b1.astype(jnp.float32).reshape(Cmid, 1)
    b2c = b2.astype(jnp.float32).reshape(Cout, 1)

    col = jnp.arange(HW, dtype=jnp.int32) % W
    cm = jnp.stack([col >= 1, col < W - 1], axis=0).astype(bf16)

    cparams = pltpu.CompilerParams(
        dimension_semantics=("arbitrary",),
        vmem_limit_bytes=_VMEM_LIMIT,
    )

    def img_spec(C):
        return pl.BlockSpec((_BLK, C, HW), lambda n: (n, 0, 0))

    def img4_spec(C):
        return pl.BlockSpec((_BLK, C, H, W), lambda n: (n, 0, 0, 0))

    def full_spec(shape):
        return pl.BlockSpec(shape, lambda n: (0,) * len(shape))

    def stat_spec(C):
        return pl.BlockSpec((_BLK, C, 1), lambda n: (n, 0, 0))

    h1, sum1, sq1 = pl.pallas_call(
        functools.partial(_conv1_body, W),
        grid=(N // _BLK,),
        in_specs=[img4_spec(Cin), full_spec((Cmid, 9 * Cin)),
                  full_spec((Cmid, 1)), full_spec((2, HW))],
        out_specs=[img_spec(Cmid), stat_spec(Cmid), stat_spec(Cmid)],
        out_shape=[jax.ShapeDtypeStruct((N, Cmid, HW), bf16),
                   jax.ShapeDtypeStruct((N, Cmid, 1), jnp.float32),
                   jax.ShapeDtypeStruct((N, Cmid, 1), jnp.float32)],
        scratch_shapes=[pltpu.VMEM((_BLK, 9 * Cin, HW), bf16)],
        compiler_params=cparams,
    )(x, w1_2d, b1c, cm)

    sc1, sh1 = _fold(sum1, sq1, gamma1, beta1, HW)

    h2, sum2, sq2 = pl.pallas_call(
        functools.partial(_conv2_body, W),
        grid=(N // _BLK,),
        in_specs=[img_spec(Cmid), full_spec((Cmid, 1)), full_spec((Cmid, 1)),
                  full_spec((Cout, 9 * Cmid)), full_spec((Cout, 1)),
                  full_spec((2, HW))],
        out_specs=[img_spec(Cout), stat_spec(Cout), stat_spec(Cout)],
        out_shape=[jax.ShapeDtypeStruct((N, Cout, HW), bf16),
                   jax.ShapeDtypeStruct((N, Cout, 1), jnp.float32),
                   jax.ShapeDtypeStruct((N, Cout, 1), jnp.float32)],
        scratch_shapes=[pltpu.VMEM((_BLK, 9 * Cmid, HW), bf16)],
        compiler_params=cparams,
    )(h1, sc1, sh1, w2_2d, b2c, cm)

    sc2, sh2 = _fold(sum2, sq2, gamma2, beta2, HW)
    y = jnp.maximum(h2.astype(jnp.float32) * sc2[None, :, :] + sh2[None, :, :], 0.0)
    return y.reshape(N, Cout, H, W)


# speedup vs baseline: 1.0519x; 1.0140x over previous
import functools

import jax
import jax.numpy as jnp
from jax.experimental import pallas as pl
from jax.experimental.pallas import tpu as pltpu

_EPS = 1e-5
_VMEM_LIMIT = 56 * 1024 * 1024
_BLK = 4


def _row_shift(src, dy, W):
    if dy == 0:
        return src
    c, HW = src.shape
    z = jnp.zeros((c, W), src.dtype)
    if dy > 0:
        return jnp.concatenate([src[:, W:], z], axis=1)
    return jnp.concatenate([z, src[:, :HW - W]], axis=1)


def _build_taps(src, cm, taps_ref, W):
    c, HW = src.shape
    for r, dy in enumerate((-1, 0, 1)):
        base = _row_shift(src, dy, W)
        tm = jnp.concatenate([base[:, HW - 1:], base[:, :HW - 1]], axis=1) * cm[0:1, :]
        tp = jnp.concatenate([base[:, 1:], base[:, :1]], axis=1) * cm[1:2, :]
        taps_ref[(3 * r) * c:(3 * r + 1) * c, :] = tm
        taps_ref[(3 * r + 1) * c:(3 * r + 2) * c, :] = base
        taps_ref[(3 * r + 2) * c:(3 * r + 3) * c, :] = tp


def _moments(h):
    return jnp.concatenate([jnp.sum(h, axis=1, keepdims=True),
                            jnp.sum(h * h, axis=1, keepdims=True)], axis=1)


def _conv1_body(W, x_ref, w_ref, b_ref, cm_ref, h1_ref, s_ref, taps):
    cin = x_ref.shape[1]
    for i in range(_BLK):
        xb = x_ref[i].astype(jnp.bfloat16).reshape(cin, -1)
        _build_taps(xb, cm_ref[...], taps.at[i], W)
        h = jnp.dot(w_ref[...], taps.at[i][...],
                    preferred_element_type=jnp.float32) + b_ref[...]
        h1_ref[i] = h.astype(jnp.bfloat16)
        s_ref[i] = _moments(h)


def _conv2_body(W, h1_ref, sc_ref, sh_ref, w_ref, b_ref, cm_ref,
                h2_ref, s_ref, taps):
    sc = sc_ref[...]
    sh = sh_ref[...]
    for i in range(_BLK):
        a32 = jnp.maximum(h1_ref[i].astype(jnp.float32) * sc + sh, 0.0)
        _build_taps(a32.astype(jnp.bfloat16), cm_ref[...], taps.at[i], W)
        h = jnp.dot(w_ref[...], taps.at[i][...],
                    preferred_element_type=jnp.float32) + b_ref[...]
        h2_ref[i] = h.astype(jnp.bfloat16)
        s_ref[i] = _moments(h)


def _fold(stats_p, gamma, beta, count):
    n = stats_p.shape[0]
    denom = jnp.float32(n * count)
    tot = jnp.sum(stats_p, axis=0) / denom
    mu = tot[:, 0]
    var = tot[:, 1] - mu * mu
    scale = gamma.astype(jnp.float32) * jax.lax.rsqrt(var + _EPS)
    shift = beta.astype(jnp.float32) - mu * scale
    return scale[:, None], shift[:, None]


def _flatten_w(w):
    cout, cin = w.shape[0], w.shape[1]
    w2d = jnp.transpose(w.astype(jnp.float32).reshape(cout, cin, 9), (0, 2, 1))
    return w2d.reshape(cout, 9 * cin).astype(jnp.bfloat16)


@jax.jit
def kernel(x, w1, b1, gamma1, beta1, w2, b2, gamma2, beta2):
    N, Cin, H, W = x.shape
    HW = H * W
    Cmid, Cout = w1.shape[0], w2.shape[0]
    bf16 = jnp.bfloat16

    w1_2d = _flatten_w(w1)
    w2_2d = _flatten_w(w2)
    b1c = b1.astype(jnp.float32).reshape(Cmid, 1)
    b2c = b2.astype(jnp.float32).reshape(Cout, 1)

    col = jnp.arange(HW, dtype=jnp.int32) % W
    cm = jnp.stack([col >= 1, col < W - 1], axis=0).astype(bf16)

    cparams = pltpu.CompilerParams(
        dimension_semantics=("arbitrary",),
        vmem_limit_bytes=_VMEM_LIMIT,
    )

    def img_spec(C):
        return pl.BlockSpec((_BLK, C, HW), lambda n: (n, 0, 0))

    def img4_spec(C):
        return pl.BlockSpec((_BLK, C, H, W), lambda n: (n, 0, 0, 0))

    def full_spec(shape):
        return pl.BlockSpec(shape, lambda n: (0,) * len(shape))

    def stat_spec(C):
        return pl.BlockSpec((_BLK, C, 2), lambda n: (n, 0, 0))

    h1, stats1 = pl.pallas_call(
        functools.partial(_conv1_body, W),
        grid=(N // _BLK,),
        in_specs=[img4_spec(Cin), full_spec((Cmid, 9 * Cin)),
                  full_spec((Cmid, 1)), full_spec((2, HW))],
        out_specs=[img_spec(Cmid), stat_spec(Cmid)],
        out_shape=[jax.ShapeDtypeStruct((N, Cmid, HW), bf16),
                   jax.ShapeDtypeStruct((N, Cmid, 2), jnp.float32)],
        scratch_shapes=[pltpu.VMEM((_BLK, 9 * Cin, HW), bf16)],
        compiler_params=cparams,
    )(x, w1_2d, b1c, cm)

    sc1, sh1 = _fold(stats1, gamma1, beta1, HW)

    h2, stats2 = pl.pallas_call(
        functools.partial(_conv2_body, W),
        grid=(N // _BLK,),
        in_specs=[img_spec(Cmid), full_spec((Cmid, 1)), full_spec((Cmid, 1)),
                  full_spec((Cout, 9 * Cmid)), full_spec((Cout, 1)),
                  full_spec((2, HW))],
        out_specs=[img_spec(Cout), stat_spec(Cout)],
        out_shape=[jax.ShapeDtypeStruct((N, Cout, HW), bf16),
                   jax.ShapeDtypeStruct((N, Cout, 2), jnp.float32)],
        scratch_shapes=[pltpu.VMEM((_BLK, 9 * Cmid, HW), bf16)],
        compiler_params=cparams,
    )(h1, sc1, sh1, w2_2d, b2c, cm)

    sc2, sh2 = _fold(stats2, gamma2, beta2, HW)
    y = jnp.maximum(h2.astype(jnp.float32) * sc2[None, :, :] + sh2[None, :, :], 0.0)
    return y.reshape(N, Cout, H, W)
